# baseline (speedup 1.0000x reference)
import functools

import jax
import jax.numpy as jnp
from jax import lax
from jax.experimental import pallas as pl
from jax.experimental.pallas import tpu as pltpu
from jax.experimental.pallas import tpu_sc as plsc

N_NODES = 10000
N_EDGES = 320000
D = 128

NC = 2
NS = 16
NW = NC * NS
PER_W = N_EDGES // NW
CHUNK = 100
N_CHUNKS = PER_W // CHUNK
IBLK = 20
N_IBLKS = N_CHUNKS // IBLK
RPS = 624
TAIL = N_NODES - NS * RPS


def _tc_pre(feature, norm, W):
    BM = 1000

    def body(f_ref, n_ref, w_ref, o_ref):
        o_ref[...] = jnp.dot(f_ref[...] * n_ref[...], w_ref[...],
                             preferred_element_type=jnp.float32)

    return pl.pallas_call(
        body,
        grid=(N_NODES // BM,),
        in_specs=[
            pl.BlockSpec((BM, D), lambda i: (i, 0)),
            pl.BlockSpec((BM, 1), lambda i: (i, 0)),
            pl.BlockSpec((D, D), lambda i: (0, 0)),
        ],
        out_specs=pl.BlockSpec((BM, D), lambda i: (i, 0)),
        out_shape=jax.ShapeDtypeStruct((N_NODES, D), jnp.float32),
    )(feature, norm, W)


def _sc_segsum(y, src_r, dst_r, zeros):
    mesh = plsc.VectorSubcoreMesh(core_axis_name="c", subcore_axis_name="s")

    @functools.partial(
        pl.kernel,
        mesh=mesh,
        out_type=jax.ShapeDtypeStruct((NC, N_NODES, D), jnp.float32),
        scratch_types=[
            pltpu.VMEM((IBLK, CHUNK), jnp.int32),
            pltpu.VMEM((IBLK, CHUNK), jnp.int32),
            pltpu.VMEM((CHUNK, D), jnp.float32),
            pltpu.VMEM((CHUNK, D), jnp.float32),
            pltpu.VMEM((CHUNK, D), jnp.float32),
            pltpu.VMEM_SHARED((N_NODES, D), jnp.float32),
            pltpu.SemaphoreType.DMA,
            pltpu.SemaphoreType.DMA,
            pltpu.SemaphoreType.DMA,
            pltpu.SemaphoreType.DMA,
            pltpu.SemaphoreType.DMA,
            pltpu.SemaphoreType.DMA,
        ],
    )
    def k(y_hbm, src_hbm, dst_hbm, z_hbm, out_hbm, src_all, dst_all,
          rows0, rows1, rows2, acc_sh, sem0, sem1, sem2, ssem0, ssem1, ssem2):
        cid = lax.axis_index("c")
        sid = lax.axis_index("s")
        wid = sid * NC + cid

        bufs = (rows0, rows1, rows2)
        sems = (sem0, sem1, sem2)
        ssems = (ssem0, ssem1, ssem2)

        def fire(c, b):
            pltpu.async_copy(y_hbm.at[src_all.at[c]], bufs[b], sems[b])

        def wait(c, b):
            pltpu.make_async_copy(y_hbm.at[src_all.at[c]], bufs[b],
                                  sems[b]).wait()

        def scat(c, b):
            pltpu.async_copy(bufs[b], acc_sh.at[dst_all.at[c]], ssems[b],
                             add=True)

        def swait(c, b):
            pltpu.make_async_copy(bufs[b], acc_sh.at[dst_all.at[c]],
                                  ssems[b]).wait()

        pltpu.sync_copy(src_hbm.at[wid, 0], src_all)
        pltpu.sync_copy(dst_hbm.at[wid, 0], dst_all)
        fire(0, 0)
        fire(1, 1)

        pltpu.sync_copy(z_hbm.at[pl.ds(0, RPS)],
                        acc_sh.at[pl.ds(sid * RPS, RPS)])

        @pl.when(sid == 0)
        def _():
            pltpu.sync_copy(z_hbm.at[pl.ds(RPS, TAIL)],
                            acc_sh.at[pl.ds(NS * RPS, TAIL)])

        plsc.subcore_barrier()

        @pl.loop(0, N_IBLKS)
        def _(blk):
            @pl.when(blk > 0)
            def _():
                pltpu.sync_copy(src_hbm.at[wid, blk], src_all)
                pltpu.sync_copy(dst_hbm.at[wid, blk], dst_all)
                fire(0, 0)
                fire(1, 1)

            @pl.loop(0, IBLK - 4, step=3)
            def _(c):
                wait(c, 0)
                scat(c, 0)

                @pl.when(c > 0)
                def _():
                    swait(c - 1, 2)

                fire(c + 2, 2)
                wait(c + 1, 1)
                scat(c + 1, 1)
                swait(c, 0)
                fire(c + 3, 0)
                wait(c + 2, 2)
                scat(c + 2, 2)
                swait(c + 1, 1)
                fire(c + 4, 1)

            wait(IBLK - 2, 0)
            scat(IBLK - 2, 0)
            wait(IBLK - 1, 1)
            scat(IBLK - 1, 1)
            swait(IBLK - 3, 2)
            swait(IBLK - 2, 0)
            swait(IBLK - 1, 1)

        plsc.subcore_barrier()
        pltpu.sync_copy(acc_sh.at[pl.ds(sid * RPS, RPS)],
                        out_hbm.at[cid, pl.ds(sid * RPS, RPS)])

        @pl.when(sid == 0)
        def _():
            pltpu.sync_copy(acc_sh.at[pl.ds(NS * RPS, TAIL)],
                            out_hbm.at[cid, pl.ds(NS * RPS, TAIL)])

    return k(y, src_r, dst_r, zeros)


def _tc_post(partials, norm, b2):
    BM = 1000

    def body(p_ref, n_ref, b_ref, o_ref):
        s = p_ref[0] + p_ref[1]
        o_ref[...] = jnp.maximum(s * n_ref[...] + b_ref[...], 0.0)

    return pl.pallas_call(
        body,
        grid=(N_NODES // BM,),
        in_specs=[
            pl.BlockSpec((NC, BM, D), lambda i: (0, i, 0)),
            pl.BlockSpec((BM, 1), lambda i: (i, 0)),
            pl.BlockSpec((1, D), lambda i: (0, 0)),
        ],
        out_specs=pl.BlockSpec((BM, D), lambda i: (i, 0)),
        out_shape=jax.ShapeDtypeStruct((N_NODES, D), jnp.float32),
    )(partials, norm, b2)


def kernel(feature, norm, edge_index, W, b):
    e = edge_index.astype(jnp.int32)
    src_r = e[0].reshape(NW, N_IBLKS, IBLK, CHUNK)
    dst_r = e[1].reshape(NW, N_IBLKS, IBLK, CHUNK)
    y = _tc_pre(feature, norm, W)
    zeros = jnp.zeros((RPS + TAIL, D), jnp.float32)
    partials = _sc_segsum(y, src_r, dst_r, zeros)
    return _tc_post(partials, norm, b.reshape(1, D))


# speedup vs baseline: 1.0280x; 1.0280x over previous
import functools

import jax
import jax.numpy as jnp
from jax import lax
from jax.experimental import pallas as pl
from jax.experimental.pallas import tpu as pltpu
from jax.experimental.pallas import tpu_sc as plsc

N_NODES = 10000
N_EDGES = 320000
D = 128

NC = 2
NS = 16
NW = NC * NS
PER_W = N_EDGES // NW
CHUNK = 100
N_CHUNKS = PER_W // CHUNK
IBLK = 20
N_IBLKS = N_CHUNKS // IBLK
RPS = 624
TAIL = N_NODES - NS * RPS


def _tc_pre(feature, norm, W):
    BM = 1000

    def body(f_ref, n_ref, w_ref, o_ref):
        o_ref[...] = jnp.dot(f_ref[...] * n_ref[...], w_ref[...],
                             preferred_element_type=jnp.float32)

    return pl.pallas_call(
        body,
        grid=(N_NODES // BM,),
        in_specs=[
            pl.BlockSpec((BM, D), lambda i: (i, 0)),
            pl.BlockSpec((BM, 1), lambda i: (i, 0)),
            pl.BlockSpec((D, D), lambda i: (0, 0)),
        ],
        out_specs=pl.BlockSpec((BM, D), lambda i: (i, 0)),
        out_shape=jax.ShapeDtypeStruct((N_NODES, D), jnp.float32),
    )(feature, norm, W)


def _sc_segsum(y, src_r, dst_r, zeros):
    mesh = plsc.VectorSubcoreMesh(core_axis_name="c", subcore_axis_name="s")

    @functools.partial(
        pl.kernel,
        mesh=mesh,
        out_type=jax.ShapeDtypeStruct((NC, N_NODES, D), jnp.float32),
        scratch_types=[
            pltpu.VMEM((IBLK, CHUNK), jnp.int32),
            pltpu.VMEM((IBLK, CHUNK), jnp.int32),
            pltpu.VMEM((CHUNK, D), jnp.float32),
            pltpu.VMEM((CHUNK, D), jnp.float32),
            pltpu.VMEM((CHUNK, D), jnp.float32),
            pltpu.VMEM_SHARED((N_NODES, D), jnp.float32),
            pltpu.SemaphoreType.DMA,
            pltpu.SemaphoreType.DMA,
            pltpu.SemaphoreType.DMA,
        ],
    )
    def k(y_hbm, src_hbm, dst_hbm, z_hbm, out_hbm, src_all, dst_all,
          rows0, rows1, rows2, acc_sh, sem0, sem1, sem2):
        cid = lax.axis_index("c")
        sid = lax.axis_index("s")
        wid = sid * NC + cid

        bufs = (rows0, rows1, rows2)
        sems = (sem0, sem1, sem2)

        def fire(c, b):
            pltpu.async_copy(y_hbm.at[src_all.at[c]], bufs[b], sems[b])

        def wait(c, b):
            pltpu.make_async_copy(y_hbm.at[src_all.at[c]], bufs[b],
                                  sems[b]).wait()

        def scat(c, b):
            pltpu.sync_copy(bufs[b], acc_sh.at[dst_all.at[c]], add=True)

        pltpu.sync_copy(src_hbm.at[wid, 0], src_all)
        pltpu.sync_copy(dst_hbm.at[wid, 0], dst_all)
        fire(0, 0)
        fire(1, 1)

        pltpu.sync_copy(z_hbm.at[pl.ds(0, RPS)],
                        acc_sh.at[pl.ds(sid * RPS, RPS)])

        @pl.when(sid == 0)
        def _():
            pltpu.sync_copy(z_hbm.at[pl.ds(RPS, TAIL)],
                            acc_sh.at[pl.ds(NS * RPS, TAIL)])

        plsc.subcore_barrier()

        @pl.loop(0, N_IBLKS)
        def _(blk):
            @pl.when(blk > 0)
            def _():
                pltpu.sync_copy(src_hbm.at[wid, blk], src_all)
                pltpu.sync_copy(dst_hbm.at[wid, blk], dst_all)
                fire(0, 0)
                fire(1, 1)

            @pl.loop(0, IBLK - 4, step=3)
            def _(c):
                fire(c + 2, 2)
                wait(c, 0)
                scat(c, 0)
                fire(c + 3, 0)
                wait(c + 1, 1)
                scat(c + 1, 1)
                fire(c + 4, 1)
                wait(c + 2, 2)
                scat(c + 2, 2)

            wait(IBLK - 2, 0)
            scat(IBLK - 2, 0)
            wait(IBLK - 1, 1)
            scat(IBLK - 1, 1)

        plsc.subcore_barrier()
        pltpu.sync_copy(acc_sh.at[pl.ds(sid * RPS, RPS)],
                        out_hbm.at[cid, pl.ds(sid * RPS, RPS)])

        @pl.when(sid == 0)
        def _():
            pltpu.sync_copy(acc_sh.at[pl.ds(NS * RPS, TAIL)],
                            out_hbm.at[cid, pl.ds(NS * RPS, TAIL)])

    return k(y, src_r, dst_r, zeros)


def _tc_post(partials, norm, b2):
    BM = 1000

    def body(p_ref, n_ref, b_ref, o_ref):
        s = p_ref[0] + p_ref[1]
        o_ref[...] = jnp.maximum(s * n_ref[...] + b_ref[...], 0.0)

    return pl.pallas_call(
        body,
        grid=(N_NODES // BM,),
        in_specs=[
            pl.BlockSpec((NC, BM, D), lambda i: (0, i, 0)),
            pl.BlockSpec((BM, 1), lambda i: (i, 0)),
            pl.BlockSpec((1, D), lambda i: (0, 0)),
        ],
        out_specs=pl.BlockSpec((BM, D), lambda i: (i, 0)),
        out_shape=jax.ShapeDtypeStruct((N_NODES, D), jnp.float32),
    )(partials, norm, b2)


def kernel(feature, norm, edge_index, W, b):
    e = edge_index.astype(jnp.int32)
    src_r = e[0].reshape(NW, N_IBLKS, IBLK, CHUNK)
    dst_r = e[1].reshape(NW, N_IBLKS, IBLK, CHUNK)
    y = _tc_pre(feature, norm, W)
    zeros = jnp.zeros((RPS + TAIL, D), jnp.float32)
    partials = _sc_segsum(y, src_r, dst_r, zeros)
    return _tc_post(partials, norm, b.reshape(1, D))
